# baseline (speedup 1.0000x reference)
import jax
import jax.numpy as jnp
from jax import lax
from jax.experimental import pallas as pl
from jax.experimental.pallas import tpu as pltpu
from jax.experimental.pallas import tpu_sc as plsc

_C = 96
_BLK = 16
_NPIX = _BLK * _BLK
_TPW = 4
_CSTRIDE = 112
_FSTRIDE = 8 * _CSTRIDE
_ACC = 3 * _FSTRIDE


def _rsqrt16(s):
    x = jnp.maximum(s, 1e-24)
    xi = plsc.bitcast(x, jnp.int32)
    y = plsc.bitcast(jnp.int32(0x5F3759DF) - (xi >> 1), jnp.float32)
    for _ in range(3):
        y = y * (1.5 - 0.5 * x * y * y)
    return y


_GDN = lax.GatherDimensionNumbers(
    offset_dims=(), collapsed_slice_dims=(0,), start_index_map=(0,))


def _splat(v, j):
    return lax.gather(v, jnp.full((16, 1), j, jnp.int32), _GDN, (1,),
                      mode=lax.GatherScatterMode.PROMISE_IN_BOUNDS)


def _sc_body(f0, f1, f2, msk, out, fbuf, mbuf, inv, acc, pout):
    wid = lax.axis_index("s") * 2 + lax.axis_index("c")
    lane = lax.iota(jnp.int32, 16)
    lane0 = lane == 0
    onev = jnp.ones((16,), jnp.float32)
    zerov = jnp.zeros((16,), jnp.float32)

    def tile_body(k, carry):
        wloss, wcnt = carry
        t = wid * _TPW + k
        img = t // 64
        tl = t % 64
        h0 = (tl // 8) * _BLK
        w0 = (tl % 8) * _BLK

        def zbody(j, _):
            acc[pl.ds(j * 16, 16)] = zerov
            return 0
        lax.fori_loop(0, _ACC // 16, zbody, 0)

        for nf, fref in enumerate((f0, f1, f2)):
            pltpu.sync_copy(
                fref.at[img, :, pl.ds(h0, _BLK), pl.ds(w0, _BLK)], fbuf)
            pltpu.sync_copy(msk.at[nf, img, tl], mbuf)

            def nbody(c, ss):
                return tuple(ss[g] + fbuf[c, g] * fbuf[c, g]
                             for g in range(16))
            ss = lax.fori_loop(
                0, _C, nbody, tuple(zerov for _ in range(16)))
            for g in range(16):
                inv[pl.ds(g * 16, 16)] = _rsqrt16(ss[g])

            base = nf * _FSTRIDE

            def rbody(g, _):
                iv_vec = inv[pl.ds(g * 16, 16)]
                m_vec = mbuf[pl.ds(g * 16, 16)]
                bvec = base + m_vec * _CSTRIDE
                gr = jnp.full((16,), g, jnp.int32)
                for j in range(16):
                    iv = _splat(iv_vec, j)
                    bj = _splat(bvec, j)
                    pc = jnp.full((16,), j, jnp.int32)
                    for cb in range(6):
                        cidx = lane + cb * 16
                        v = plsc.load_gather(fbuf, [cidx, gr, pc])
                        plsc.addupdate_scatter(acc, [bj + cidx], v * iv)
                    plsc.addupdate_scatter(acc, [bj + _C], onev, mask=lane0)
                return 0
            lax.fori_loop(0, _BLK, rbody, 0)

        nclass = zerov
        tsum = zerov
        for u in range(8):
            b = u * _CSTRIDE
            cv0 = acc[pl.ds(b + _C, 16)]
            cv1 = acc[pl.ds(_FSTRIDE + b + _C, 16)]
            cv2 = acc[pl.ds(2 * _FSTRIDE + b + _C, 16)]
            pres = (cv0 > 0) & (cv1 > 0) & (cv2 > 0)
            r0 = _splat(1.0 / jnp.maximum(cv0, 1.0), 0)
            r1 = _splat(1.0 / jnp.maximum(cv1, 1.0), 0)
            r2 = _splat(1.0 / jnp.maximum(cv2, 1.0), 0)
            dsum = zerov
            for cb in range(6):
                p0 = acc[pl.ds(b + cb * 16, 16)]
                p1 = acc[pl.ds(_FSTRIDE + b + cb * 16, 16)]
                p2 = acc[pl.ds(2 * _FSTRIDE + b + cb * 16, 16)]
                dsum = dsum + jnp.abs(p0 * r0 - 2.0 * (p1 * r1) + p2 * r2)
            tu = jnp.sum(dsum) * (1.0 / _C)
            presv = jnp.where(pres, 1.0, 0.0).astype(jnp.float32)
            nclass = nclass + presv
            tsum = tsum + tu * presv
        lt = tsum / jnp.maximum(nclass, 1.0)
        hasv = jnp.where(nclass > 0, 1.0, 0.0).astype(jnp.float32)
        return wloss + lt * hasv, wcnt + hasv

    wloss, wcnt = lax.fori_loop(0, _TPW, tile_body, (zerov, zerov))

    pout[...] = (jnp.where(lane == 0, wloss, 0.0)
                 + jnp.where(lane == 1, _splat(wcnt, 0), 0.0)
                 ).astype(jnp.float32)
    pltpu.sync_copy(pout, out.at[wid // 8, pl.ds((wid % 8) * 16, 16)])


def _final_kernel(x_ref, o_ref):
    x = x_ref[...]
    col = lax.broadcasted_iota(jnp.int32, (4, 128), 1)
    lsum = jnp.sum(jnp.where(col % 16 == 0, x, 0.0))
    csum = jnp.sum(jnp.where(col % 16 == 1, x, 0.0))
    final = jnp.where(csum > 0, lsum / jnp.maximum(csum, 1.0), lsum)
    r = lax.broadcasted_iota(jnp.int32, (8, 128), 0)
    c = lax.broadcasted_iota(jnp.int32, (8, 128), 1)
    o_ref[...] = jnp.where((r == 0) & (c == 0), final, 0.0)


def kernel(feat0, feat1, feat2, mask0, mask1, mask2):
    msk = jnp.stack([
        m.astype(jnp.int32).reshape(2, 8, 16, 8, 16)
        .transpose(0, 1, 3, 2, 4).reshape(2, 64, _NPIX)
        for m in (mask0, mask1, mask2)])

    sc = pl.kernel(
        _sc_body,
        mesh=plsc.VectorSubcoreMesh(core_axis_name="c", subcore_axis_name="s"),
        out_type=jax.ShapeDtypeStruct((4, 128), jnp.float32),
        scratch_types=[
            pltpu.VMEM((_C, _BLK, _BLK), jnp.float32),
            pltpu.VMEM((_NPIX,), jnp.int32),
            pltpu.VMEM((_NPIX,), jnp.float32),
            pltpu.VMEM((_ACC,), jnp.float32),
            pltpu.VMEM((16,), jnp.float32),
        ],
        compiler_params=pltpu.CompilerParams(
            use_tc_tiling_on_sc=False, needs_layout_passes=False),
    )
    partial = sc(feat0, feat1, feat2, msk)

    out = pl.pallas_call(
        _final_kernel,
        out_specs=pl.BlockSpec((8, 128), lambda: (0, 0)),
        out_shape=jax.ShapeDtypeStruct((8, 128), jnp.float32),
    )(partial)
    return out[0, 0]


# speedup vs baseline: 1.3426x; 1.3426x over previous
import jax
import jax.numpy as jnp
from jax import lax
from jax.experimental import pallas as pl
from jax.experimental.pallas import tpu as pltpu
from jax.experimental.pallas import tpu_sc as plsc

_C = 96
_BLK = 16
_NPIX = _BLK * _BLK
_TPW = 4
_CSTRIDE = 112
_FSTRIDE = 8 * _CSTRIDE
_ACC = 3 * _FSTRIDE


def _rsqrt16(s):
    x = jnp.maximum(s, 1e-24)
    xi = plsc.bitcast(x, jnp.int32)
    y = plsc.bitcast(jnp.int32(0x5F3759DF) - (xi >> 1), jnp.float32)
    for _ in range(3):
        y = y * (1.5 - 0.5 * x * y * y)
    return y


_GDN = lax.GatherDimensionNumbers(
    offset_dims=(), collapsed_slice_dims=(0,), start_index_map=(0,))


def _splat(v, j):
    return lax.gather(v, jnp.full((16, 1), j, jnp.int32), _GDN, (1,),
                      mode=lax.GatherScatterMode.PROMISE_IN_BOUNDS)


def _sc_body(f0, f1, f2, msk, out, fb0, fb1, fb2, mbuf, inv, acc, pout,
             sem0, sem1, sem2):
    wid = lax.axis_index("s") * 2 + lax.axis_index("c")
    lane = lax.iota(jnp.int32, 16)
    lane0 = lane == 0
    onev = jnp.ones((16,), jnp.float32)
    zerov = jnp.zeros((16,), jnp.float32)

    def tile_body(k, carry):
        wloss, wcnt = carry
        t = wid * _TPW + k
        img = t // 64
        tl = t % 64
        h0 = (tl // 8) * _BLK
        w0 = (tl % 8) * _BLK

        copies = [
            pltpu.async_copy(
                fref.at[img, :, pl.ds(h0, _BLK), pl.ds(w0, _BLK)], fb, sem)
            for fref, fb, sem in ((f0, fb0, sem0), (f1, fb1, sem1),
                                  (f2, fb2, sem2))]

        @plsc.parallel_loop(0, _ACC // 16)
        def _zero(j):
            acc[pl.ds(j * 16, 16)] = zerov

        for nf, (fbuf, cp) in enumerate(zip((fb0, fb1, fb2), copies)):
            cp.wait()
            pltpu.sync_copy(msk.at[nf, img, tl], mbuf)

            def nbody(c, ss):
                return tuple(ss[g] + fbuf[c, g] * fbuf[c, g]
                             for g in range(16))
            ss = lax.fori_loop(
                0, _C, nbody, tuple(zerov for _ in range(16)))
            for g in range(16):
                inv[pl.ds(g * 16, 16)] = _rsqrt16(ss[g])

            base = nf * _FSTRIDE

            @plsc.parallel_loop(0, _BLK, unroll=2)
            def _rows(g):
                iv_vec = inv[pl.ds(g * 16, 16)]
                m_vec = mbuf[pl.ds(g * 16, 16)]
                bvec = base + m_vec * _CSTRIDE
                gr = jnp.full((16,), g, jnp.int32)
                for j in range(16):
                    iv = _splat(iv_vec, j)
                    bj = _splat(bvec, j)
                    pc = jnp.full((16,), j, jnp.int32)
                    for cb in range(6):
                        cidx = lane + cb * 16
                        v = plsc.load_gather(fbuf, [cidx, gr, pc])
                        plsc.addupdate_scatter(acc, [bj + cidx], v * iv)
                    plsc.addupdate_scatter(acc, [bj + _C], onev, mask=lane0)

        nclass = zerov
        tsum = zerov
        for u in range(8):
            b = u * _CSTRIDE
            cv0 = acc[pl.ds(b + _C, 16)]
            cv1 = acc[pl.ds(_FSTRIDE + b + _C, 16)]
            cv2 = acc[pl.ds(2 * _FSTRIDE + b + _C, 16)]
            pres = (cv0 > 0) & (cv1 > 0) & (cv2 > 0)
            r0 = _splat(1.0 / jnp.maximum(cv0, 1.0), 0)
            r1 = _splat(1.0 / jnp.maximum(cv1, 1.0), 0)
            r2 = _splat(1.0 / jnp.maximum(cv2, 1.0), 0)
            dsum = zerov
            for cb in range(6):
                p0 = acc[pl.ds(b + cb * 16, 16)]
                p1 = acc[pl.ds(_FSTRIDE + b + cb * 16, 16)]
                p2 = acc[pl.ds(2 * _FSTRIDE + b + cb * 16, 16)]
                dsum = dsum + jnp.abs(p0 * r0 - 2.0 * (p1 * r1) + p2 * r2)
            tu = jnp.sum(dsum) * (1.0 / _C)
            presv = jnp.where(pres, 1.0, 0.0).astype(jnp.float32)
            nclass = nclass + presv
            tsum = tsum + tu * presv
        lt = tsum / jnp.maximum(nclass, 1.0)
        hasv = jnp.where(nclass > 0, 1.0, 0.0).astype(jnp.float32)
        return wloss + lt * hasv, wcnt + hasv

    wloss, wcnt = lax.fori_loop(0, _TPW, tile_body, (zerov, zerov))

    pout[...] = (jnp.where(lane == 0, wloss, 0.0)
                 + jnp.where(lane == 1, _splat(wcnt, 0), 0.0)
                 ).astype(jnp.float32)
    pltpu.sync_copy(pout, out.at[wid // 8, pl.ds((wid % 8) * 16, 16)])


def _final_kernel(x_ref, o_ref):
    x = x_ref[...]
    col = lax.broadcasted_iota(jnp.int32, (4, 128), 1)
    lsum = jnp.sum(jnp.where(col % 16 == 0, x, 0.0))
    csum = jnp.sum(jnp.where(col % 16 == 1, x, 0.0))
    final = jnp.where(csum > 0, lsum / jnp.maximum(csum, 1.0), lsum)
    r = lax.broadcasted_iota(jnp.int32, (8, 128), 0)
    c = lax.broadcasted_iota(jnp.int32, (8, 128), 1)
    o_ref[...] = jnp.where((r == 0) & (c == 0), final, 0.0)


def kernel(feat0, feat1, feat2, mask0, mask1, mask2):
    msk = jnp.stack([
        m.astype(jnp.int32).reshape(2, 8, 16, 8, 16)
        .transpose(0, 1, 3, 2, 4).reshape(2, 64, _NPIX)
        for m in (mask0, mask1, mask2)])

    sc = pl.kernel(
        _sc_body,
        mesh=plsc.VectorSubcoreMesh(core_axis_name="c", subcore_axis_name="s"),
        out_type=jax.ShapeDtypeStruct((4, 128), jnp.float32),
        scratch_types=[
            pltpu.VMEM((_C, _BLK, _BLK), jnp.float32),
            pltpu.VMEM((_C, _BLK, _BLK), jnp.float32),
            pltpu.VMEM((_C, _BLK, _BLK), jnp.float32),
            pltpu.VMEM((_NPIX,), jnp.int32),
            pltpu.VMEM((_NPIX,), jnp.float32),
            pltpu.VMEM((_ACC,), jnp.float32),
            pltpu.VMEM((16,), jnp.float32),
            pltpu.SemaphoreType.DMA,
            pltpu.SemaphoreType.DMA,
            pltpu.SemaphoreType.DMA,
        ],
        compiler_params=pltpu.CompilerParams(
            use_tc_tiling_on_sc=False, needs_layout_passes=False),
    )
    partial = sc(feat0, feat1, feat2, msk)

    out = pl.pallas_call(
        _final_kernel,
        out_specs=pl.BlockSpec((8, 128), lambda: (0, 0)),
        out_shape=jax.ShapeDtypeStruct((8, 128), jnp.float32),
    )(partial)
    return out[0, 0]
